# baseline (speedup 1.0000x reference)
import functools

import jax
import jax.numpy as jnp
from jax.experimental import pallas as pl
from jax.experimental.pallas import tpu as pltpu

_BF = jnp.bfloat16
_DN = ('NHWC', 'HWIO', 'NHWC')


def _vq_body(zp_ref, cb_ref, q_ref):
    zp = zp_ref[...]
    cb = cb_ref[...]
    dots = jax.lax.dot_general(zp, cb, (((1,), (1,)), ((), ())),
                               preferred_element_type=jnp.float32)
    d = (jnp.sum(zp * zp, axis=1, keepdims=True)
         + jnp.sum(cb * cb, axis=1)[None, :]
         - 2.0 * dots)
    idx = jnp.argmin(d, axis=1)
    onehot = (jax.lax.broadcasted_iota(jnp.int32, d.shape, 1)
              == idx[:, None]).astype(jnp.float32)
    q_ref[...] = jnp.dot(onehot, cb, preferred_element_type=jnp.float32)


def _vq_quantize(zp, codebook):
    return pl.pallas_call(
        _vq_body,
        out_shape=jax.ShapeDtypeStruct(zp.shape, jnp.float32),
    )(zp, codebook)


def _deconv_body(x_ref, w_ref, b_ref, o_ref, acc_ref, *, Th, W, Wp, Co):
    X = x_ref[0, 0]
    bias = b_ref[...].astype(jnp.float32)
    for u in range(4):
        a, j = u % 2, u // 2
        Wu = w_ref[:, u * 4 * Co:(u + 1) * 4 * Co]
        Zu = jnp.dot(X, Wu, preferred_element_type=jnp.float32)
        Zr = Zu.reshape(Th + 2, Wp, 4 * Co)
        for v in range(4):
            t, l = v % 2, v // 2
            term = Zr[j + a:j + a + Th, l + t:l + t + W, v * Co:(v + 1) * Co]
            p = 2 * a + t
            if j == 0 and l == 0:
                acc_ref[p] = bias[None] + term
            else:
                acc_ref[p] += term
    for p in range(4):
        o_ref[0, 0, p] = jnp.maximum(acc_ref[p], 0.2 * acc_ref[p]).astype(_BF)


def _deconv_pallas(x, w, b):
    B, H, W, Ci = x.shape
    Co = w.shape[-1]
    S = 1 if H <= 56 else 4
    Th = H // S
    Wp = -(-(W + 2) // 8) * 8
    xp = jnp.pad(x.astype(_BF), ((0, 0), (1, 1), (1, Wp - W - 1), (0, 0)))
    xs = jnp.stack([xp[:, s * Th:s * Th + Th + 2] for s in range(S)], axis=1)
    xs = xs.reshape(B, S, (Th + 2) * Wp, Ci)
    wc = jnp.transpose(w.astype(_BF), (2, 0, 1, 3)).reshape(Ci, 16 * Co)
    body = functools.partial(_deconv_body, Th=Th, W=W, Wp=Wp, Co=Co)
    out = pl.pallas_call(
        body,
        grid=(B, S),
        in_specs=[
            pl.BlockSpec((1, 1, (Th + 2) * Wp, Ci), lambda i, s: (i, s, 0, 0)),
            pl.BlockSpec((Ci, 16 * Co), lambda i, s: (0, 0)),
            pl.BlockSpec((1, Co), lambda i, s: (0, 0)),
        ],
        out_specs=pl.BlockSpec((1, 1, 4, Th, W, Co),
                               lambda i, s: (i, s, 0, 0, 0, 0)),
        out_shape=jax.ShapeDtypeStruct((B, S, 4, Th, W, Co), _BF),
        scratch_shapes=[pltpu.VMEM((4, Th, W, Co), jnp.float32)],
    )(xs, wc, b.reshape(1, Co))
    y = out.reshape(B, S, 2, 2, Th, W, Co)
    y = jnp.transpose(y, (0, 1, 4, 2, 5, 3, 6))
    return y.reshape(B, 2 * H, 2 * W, Co)


def _conv(x, w, b, pad):
    y = jax.lax.conv_general_dilated(x.astype(_BF), w.astype(_BF), (1, 1),
                                     ((pad, pad), (pad, pad)),
                                     dimension_numbers=_DN,
                                     preferred_element_type=jnp.float32)
    return y + b[None, None, None, :]


def _maxpool(x, p):
    return jax.lax.reduce_window(x, -jnp.inf, jax.lax.max, (1, p, p, 1),
                                 (1, p, p, 1), 'VALID')


def _lrelu(x):
    return jax.nn.leaky_relu(x, 0.2)


def kernel(input, enc_params, dec_deconv, dec_conv, codebook):
    pools = [2, 2, 2, 2, 0]
    h = jnp.transpose(input, (0, 2, 3, 1))
    n = len(enc_params)
    for i, (w, b) in enumerate(enc_params):
        k = w.shape[0]
        h = _conv(h, w, b, k // 2)
        if pools[i] > 0:
            h = _maxpool(h, pools[i])
        h = _lrelu(h) if i < n - 1 else jax.nn.sigmoid(h)

    B, H, W, C = h.shape
    zp = h.reshape(-1, C)
    q = _vq_quantize(zp, codebook)
    qz = q.reshape(B, H, W, C)

    for (w, b) in dec_deconv:
        qz = _deconv_pallas(qz, w, b)
    w, b = dec_conv[0]
    qz = _lrelu(_conv(qz, w, b, 1))
    w, b = dec_conv[1]
    qz = jax.nn.sigmoid(_conv(qz, w, b, 0))
    return jnp.transpose(qz, (0, 3, 1, 2))
